# Initial kernel scaffold; baseline (speedup 1.0000x reference)
import jax
import jax.numpy as jnp
from jax.experimental import pallas as pl


def kernel(x0, current_profile, w1, b1, w2, b2):
    raise NotImplementedError("write your pallas kernel here")


# speedup vs baseline: 1.0230x; 1.0230x over previous
import jax
import jax.numpy as jnp
from jax import lax
from jax.experimental import pallas as pl
from jax.experimental.pallas import tpu as pltpu

_NUM_STEPS = 50
_DT = (1.0 - 0.0) / _NUM_STEPS
_SEL = 64
_CHUNK = 1024


def _gather_kernel(z_ref, cur_ref, csel_ref):
    T = cur_ref.shape[1]
    z = z_ref[0]
    kk = lax.broadcasted_iota(jnp.int32, (T, _SEL), 0).astype(jnp.float32)
    jj = (lax.broadcasted_iota(jnp.int32, (T, _SEL), 1) + z
          ).astype(jnp.float32)
    ts = (jj * jnp.float32(_DT)) * jnp.float32(T - 1)
    sel = jnp.logical_and(ts >= kk, ts < kk + 1.0).astype(jnp.float32)
    csel_ref[...] = jnp.dot(cur_ref[...], sel,
                            preferred_element_type=jnp.float32)


def _euler_kernel(x_ref, csel_ref, w1x_ref, w1c_ref, w1t_ref, b1_ref,
                  w2_ref, b2_ref, out_ref):
    n = pl.program_id(0)
    dt = jnp.float32(_DT)
    t_n = dt * n.astype(jnp.float32)
    tb = t_n * w1t_ref[...] + b1_ref[...]
    lane = lax.broadcasted_iota(jnp.int32, (1, _SEL), 1)
    mask = (lane == n).astype(jnp.float32)

    B = x_ref.shape[0]
    for c in range(B // _CHUNK):
        rows = slice(c * _CHUNK, (c + 1) * _CHUNK)
        x = x_ref[rows, :]
        cur_col = jnp.sum(csel_ref[rows, :] * mask, axis=1, keepdims=True)
        h = jnp.dot(x, w1x_ref[...], preferred_element_type=jnp.float32)
        h = jnp.tanh(h + cur_col * w1c_ref[...] + tb)
        dx = jnp.dot(h, w2_ref[...], preferred_element_type=jnp.float32)
        new_x = x + dt * (dx + b2_ref[...])
        x_ref[rows, :] = new_x
        out_ref[rows, :] = new_x


def kernel(x0, current_profile, w1, b1, w2, b2):
    B, state_dim = x0.shape
    T = current_profile.shape[1]
    H = w1.shape[1]
    cur_bt = current_profile.reshape(B, T)
    w1x = w1[:state_dim]
    w1c = w1[state_dim:state_dim + 1]
    w1t = w1[state_dim + 1:state_dim + 2]

    vm = pl.BlockSpec(memory_space=pltpu.MemorySpace.VMEM)
    zero = jnp.zeros((1,), jnp.int32)
    csel = pl.pallas_call(
        _gather_kernel,
        out_shape=jax.ShapeDtypeStruct((B, _SEL), jnp.float32),
        in_specs=[pl.BlockSpec(memory_space=pltpu.MemorySpace.SMEM), vm],
        out_specs=vm,
    )(zero, cur_bt)

    return pl.pallas_call(
        _euler_kernel,
        out_shape=jax.ShapeDtypeStruct((B, state_dim), x0.dtype),
        grid=(_NUM_STEPS,),
        in_specs=[vm] * 8,
        out_specs=vm,
        compiler_params=pltpu.CompilerParams(
            dimension_semantics=("arbitrary",),
            vmem_limit_bytes=48 * 1024 * 1024),
    )(x0, csel, w1x, w1c, w1t, b1, w2, b2)


# speedup vs baseline: 1.0261x; 1.0030x over previous
import jax
import jax.numpy as jnp
from jax import lax
from jax.experimental import pallas as pl
from jax.experimental.pallas import tpu as pltpu

_NUM_STEPS = 50
_DT = (1.0 - 0.0) / _NUM_STEPS
_SEL = 64
_CHUNK = 1024


def _init_kernel(z_ref, cur_ref, x0_ref, csel_ref, xb_ref):
    T = cur_ref.shape[1]
    z = z_ref[0]
    kk = lax.broadcasted_iota(jnp.int32, (T, _SEL), 0).astype(jnp.float32)
    jj = (lax.broadcasted_iota(jnp.int32, (T, _SEL), 1) + z
          ).astype(jnp.float32)
    ts = (jj * jnp.float32(_DT)) * jnp.float32(T - 1)
    sel = jnp.logical_and(ts >= kk, ts < kk + 1.0).astype(jnp.float32)
    csel_ref[...] = jnp.dot(cur_ref[...], sel,
                            preferred_element_type=jnp.float32)
    xb_ref[...] = x0_ref[...].astype(jnp.bfloat16)


def _mm(a, b):
    return lax.dot_general(a, b, (((1,), (0,)), ((), ())),
                           preferred_element_type=jnp.float32)


def _euler_kernel(x_ref, xb_ref, csel_ref, w1x_ref, w1c_ref, w1t_ref,
                  b1_ref, w2dt_ref, b2dt_ref, out_ref):
    n = pl.program_id(0)
    dt = jnp.float32(_DT)
    t_n = dt * n.astype(jnp.float32)
    tb = (t_n * w1t_ref[...] + b1_ref[...]).astype(jnp.bfloat16)
    lane = lax.broadcasted_iota(jnp.int32, (1, _SEL), 1)
    mask = (lane == n).astype(jnp.float32)

    B = x_ref.shape[0]
    for c in range(B // _CHUNK):
        rows = slice(c * _CHUNK, (c + 1) * _CHUNK)
        x = x_ref[rows, :]
        cur_col = jnp.sum(csel_ref[rows, :] * mask, axis=1, keepdims=True)
        h = _mm(xb_ref[rows, :], w1x_ref[...]).astype(jnp.bfloat16)
        h = jnp.tanh(h + cur_col.astype(jnp.bfloat16) * w1c_ref[...] + tb)
        dxs = _mm(h, w2dt_ref[...])
        new_x = x + (dxs + b2dt_ref[...])
        x_ref[rows, :] = new_x
        xb_ref[rows, :] = new_x.astype(jnp.bfloat16)
        out_ref[rows, :] = new_x


def kernel(x0, current_profile, w1, b1, w2, b2):
    B, state_dim = x0.shape
    T = current_profile.shape[1]
    H = w1.shape[1]
    cur_bt = current_profile.reshape(B, T)
    dt = jnp.float32(_DT)
    w1x = w1[:state_dim].astype(jnp.bfloat16)
    w1c = w1[state_dim:state_dim + 1].astype(jnp.bfloat16)
    w1t = w1[state_dim + 1:state_dim + 2]
    w2dt = (dt * w2).astype(jnp.bfloat16)
    b2dt = dt * b2

    vm = pl.BlockSpec(memory_space=pltpu.MemorySpace.VMEM)
    zero = jnp.zeros((1,), jnp.int32)
    csel, xb0 = pl.pallas_call(
        _init_kernel,
        out_shape=(jax.ShapeDtypeStruct((B, _SEL), jnp.float32),
                   jax.ShapeDtypeStruct((B, state_dim), jnp.bfloat16)),
        in_specs=[pl.BlockSpec(memory_space=pltpu.MemorySpace.SMEM), vm, vm],
        out_specs=(vm, vm),
    )(zero, cur_bt, x0)

    return pl.pallas_call(
        _euler_kernel,
        out_shape=jax.ShapeDtypeStruct((B, state_dim), x0.dtype),
        grid=(_NUM_STEPS,),
        in_specs=[vm] * 9,
        out_specs=vm,
        compiler_params=pltpu.CompilerParams(
            dimension_semantics=("arbitrary",),
            vmem_limit_bytes=48 * 1024 * 1024),
    )(x0, xb0, csel, w1x, w1c, w1t, b1, w2dt, b2dt)


# speedup vs baseline: 1.0618x; 1.0348x over previous
import jax
import jax.numpy as jnp
from jax import lax
from jax.experimental import pallas as pl
from jax.experimental.pallas import tpu as pltpu

_NUM_STEPS = 50
_DT = (1.0 - 0.0) / _NUM_STEPS
_SEL = 64
_CHUNK = 1024


def _mm(a, b):
    return lax.dot_general(a, b, (((1,), (0,)), ((), ())),
                           preferred_element_type=jnp.float32)


def _euler_kernel(x_ref, cur_ref, w1x_ref, w1c_ref, w1t_ref,
                  b1_ref, w2dt_ref, b2dt_ref, out_ref, csel_ref, xb_ref):
    n = pl.program_id(0)
    dt = jnp.float32(_DT)
    T = cur_ref.shape[1]

    @pl.when(n == 0)
    def _():
        kk = lax.broadcasted_iota(jnp.int32, (T, _SEL), 0).astype(jnp.float32)
        jj = (lax.broadcasted_iota(jnp.int32, (T, _SEL), 1) + n
              ).astype(jnp.float32)
        ts = (jj * dt) * jnp.float32(T - 1)
        sel = jnp.logical_and(ts >= kk, ts < kk + 1.0).astype(jnp.float32)
        csel_ref[...] = jnp.dot(cur_ref[...], sel,
                                preferred_element_type=jnp.float32)
        xb_ref[...] = x_ref[...].astype(jnp.bfloat16)

    t_n = dt * n.astype(jnp.float32)
    tb = (t_n * w1t_ref[...] + b1_ref[...]).astype(jnp.bfloat16)
    lane = lax.broadcasted_iota(jnp.int32, (1, _SEL), 1)
    mask = (lane == n).astype(jnp.float32)

    B = x_ref.shape[0]
    for c in range(B // _CHUNK):
        rows = slice(c * _CHUNK, (c + 1) * _CHUNK)
        x = x_ref[rows, :]
        cur_col = jnp.sum(csel_ref[rows, :] * mask, axis=1, keepdims=True)
        h = _mm(xb_ref[rows, :], w1x_ref[...]).astype(jnp.bfloat16)
        h = jnp.tanh(h + cur_col.astype(jnp.bfloat16) * w1c_ref[...] + tb)
        dxs = _mm(h, w2dt_ref[...])
        new_x = x + (dxs + b2dt_ref[...])
        x_ref[rows, :] = new_x
        xb_ref[rows, :] = new_x.astype(jnp.bfloat16)
        out_ref[rows, :] = new_x


def kernel(x0, current_profile, w1, b1, w2, b2):
    B, state_dim = x0.shape
    T = current_profile.shape[1]
    H = w1.shape[1]
    cur_bt = current_profile.reshape(B, T)
    dt = jnp.float32(_DT)
    w1x = w1[:state_dim].astype(jnp.bfloat16)
    w1c = w1[state_dim:state_dim + 1].astype(jnp.bfloat16)
    w1t = w1[state_dim + 1:state_dim + 2]
    w2dt = (dt * w2).astype(jnp.bfloat16)
    b2dt = dt * b2

    vm = pl.BlockSpec(memory_space=pltpu.MemorySpace.VMEM)
    return pl.pallas_call(
        _euler_kernel,
        out_shape=jax.ShapeDtypeStruct((B, state_dim), x0.dtype),
        grid=(_NUM_STEPS,),
        in_specs=[vm] * 8,
        out_specs=vm,
        scratch_shapes=[pltpu.VMEM((B, _SEL), jnp.float32),
                        pltpu.VMEM((B, state_dim), jnp.bfloat16)],
        compiler_params=pltpu.CompilerParams(
            dimension_semantics=("arbitrary",),
            vmem_limit_bytes=48 * 1024 * 1024),
    )(x0, cur_bt, w1x, w1c, w1t, b1, w2dt, b2dt)


# speedup vs baseline: 1.0960x; 1.0322x over previous
import jax
import jax.numpy as jnp
from jax import lax
from jax.experimental import pallas as pl
from jax.experimental.pallas import tpu as pltpu

_NUM_STEPS = 50
_DT = (1.0 - 0.0) / _NUM_STEPS
_SEL = 64
_CHUNK = 1024
_UNROLL = 2


def _mm(a, b):
    return lax.dot_general(a, b, (((1,), (0,)), ((), ())),
                           preferred_element_type=jnp.float32)


def _euler_kernel(x_ref, cur_ref, w1x_ref, w1c_ref, w1t_ref,
                  b1_ref, w2dt_ref, b2dt_ref, out_ref, csel_ref, xb_ref):
    n = pl.program_id(0)
    dt = jnp.float32(_DT)
    T = cur_ref.shape[1]
    B = x_ref.shape[0]

    @pl.when(n == 0)
    def _():
        kk = lax.broadcasted_iota(jnp.int32, (T, _SEL), 0).astype(jnp.float32)
        jj = (lax.broadcasted_iota(jnp.int32, (T, _SEL), 1) + n
              ).astype(jnp.float32)
        ts = (jj * dt) * jnp.float32(T - 1)
        sel = jnp.logical_and(ts >= kk, ts < kk + 1.0).astype(jnp.float32)
        csel_ref[...] = jnp.dot(cur_ref[...], sel,
                                preferred_element_type=jnp.float32)
        xb_ref[...] = x_ref[...].astype(jnp.bfloat16)

    lane = lax.broadcasted_iota(jnp.int32, (1, _SEL), 1)
    for s in range(_UNROLL):
        n_s = n * _UNROLL + s
        t_n = dt * n_s.astype(jnp.float32)
        tb = (t_n * w1t_ref[...] + b1_ref[...]).astype(jnp.bfloat16)
        mask = (lane == n_s).astype(jnp.float32)
        for c in range(B // _CHUNK):
            rows = slice(c * _CHUNK, (c + 1) * _CHUNK)
            x = x_ref[rows, :]
            cur_col = jnp.sum(csel_ref[rows, :] * mask, axis=1,
                              keepdims=True)
            h = _mm(xb_ref[rows, :], w1x_ref[...]).astype(jnp.bfloat16)
            h = jnp.tanh(h + cur_col.astype(jnp.bfloat16) * w1c_ref[...]
                         + tb)
            dxs = _mm(h, w2dt_ref[...])
            new_x = x + (dxs + b2dt_ref[...])
            x_ref[rows, :] = new_x
            xb_ref[rows, :] = new_x.astype(jnp.bfloat16)
            out_ref[rows, :] = new_x


def kernel(x0, current_profile, w1, b1, w2, b2):
    B, state_dim = x0.shape
    T = current_profile.shape[1]
    H = w1.shape[1]
    cur_bt = current_profile.reshape(B, T)
    dt = jnp.float32(_DT)
    w1x = w1[:state_dim].astype(jnp.bfloat16)
    w1c = w1[state_dim:state_dim + 1].astype(jnp.bfloat16)
    w1t = w1[state_dim + 1:state_dim + 2]
    w2dt = (dt * w2).astype(jnp.bfloat16)
    b2dt = dt * b2

    vm = pl.BlockSpec(memory_space=pltpu.MemorySpace.VMEM)
    return pl.pallas_call(
        _euler_kernel,
        out_shape=jax.ShapeDtypeStruct((B, state_dim), x0.dtype),
        grid=(_NUM_STEPS // _UNROLL,),
        in_specs=[vm] * 8,
        out_specs=vm,
        scratch_shapes=[pltpu.VMEM((B, _SEL), jnp.float32),
                        pltpu.VMEM((B, state_dim), jnp.bfloat16)],
        compiler_params=pltpu.CompilerParams(
            dimension_semantics=("arbitrary",),
            vmem_limit_bytes=48 * 1024 * 1024),
    )(x0, cur_bt, w1x, w1c, w1t, b1, w2dt, b2dt)


# speedup vs baseline: 1.1215x; 1.0233x over previous
import jax
import jax.numpy as jnp
from jax import lax
from jax.experimental import pallas as pl
from jax.experimental.pallas import tpu as pltpu

_NUM_STEPS = 50
_DT = (1.0 - 0.0) / _NUM_STEPS
_SEL = 64
_CHUNK = 1024
_UNROLL = 5


def _mm(a, b):
    return lax.dot_general(a, b, (((1,), (0,)), ((), ())),
                           preferred_element_type=jnp.float32)


def _euler_kernel(x_ref, cur_ref, w1x_ref, w1c_ref, w1t_ref,
                  b1_ref, w2dt_ref, b2dt_ref, out_ref, csel_ref, xb_ref):
    n = pl.program_id(0)
    dt = jnp.float32(_DT)
    T = cur_ref.shape[1]
    B = x_ref.shape[0]

    @pl.when(n == 0)
    def _():
        kk = lax.broadcasted_iota(jnp.int32, (T, _SEL), 0).astype(jnp.float32)
        jj = (lax.broadcasted_iota(jnp.int32, (T, _SEL), 1) + n
              ).astype(jnp.float32)
        ts = (jj * dt) * jnp.float32(T - 1)
        sel = jnp.logical_and(ts >= kk, ts < kk + 1.0).astype(jnp.float32)
        csel_ref[...] = jnp.dot(cur_ref[...], sel,
                                preferred_element_type=jnp.float32)
        xb_ref[...] = x_ref[...].astype(jnp.bfloat16)

    lane = lax.broadcasted_iota(jnp.int32, (1, _SEL), 1)
    for s in range(_UNROLL):
        n_s = n * _UNROLL + s
        t_n = dt * n_s.astype(jnp.float32)
        tb = (t_n * w1t_ref[...] + b1_ref[...]).astype(jnp.bfloat16)
        mask = (lane == n_s).astype(jnp.float32)
        for c in range(B // _CHUNK):
            rows = slice(c * _CHUNK, (c + 1) * _CHUNK)
            x = x_ref[rows, :]
            cur_col = jnp.sum(csel_ref[rows, :] * mask, axis=1,
                              keepdims=True)
            h = _mm(xb_ref[rows, :], w1x_ref[...]).astype(jnp.bfloat16)
            h = jnp.tanh(h + cur_col.astype(jnp.bfloat16) * w1c_ref[...]
                         + tb)
            dxs = _mm(h, w2dt_ref[...])
            new_x = x + (dxs + b2dt_ref[...])
            x_ref[rows, :] = new_x
            xb_ref[rows, :] = new_x.astype(jnp.bfloat16)
            out_ref[rows, :] = new_x


def kernel(x0, current_profile, w1, b1, w2, b2):
    B, state_dim = x0.shape
    T = current_profile.shape[1]
    H = w1.shape[1]
    cur_bt = current_profile.reshape(B, T)
    dt = jnp.float32(_DT)
    w1x = w1[:state_dim].astype(jnp.bfloat16)
    w1c = w1[state_dim:state_dim + 1].astype(jnp.bfloat16)
    w1t = w1[state_dim + 1:state_dim + 2]
    w2dt = (dt * w2).astype(jnp.bfloat16)
    b2dt = dt * b2

    vm = pl.BlockSpec(memory_space=pltpu.MemorySpace.VMEM)
    return pl.pallas_call(
        _euler_kernel,
        out_shape=jax.ShapeDtypeStruct((B, state_dim), x0.dtype),
        grid=(_NUM_STEPS // _UNROLL,),
        in_specs=[vm] * 8,
        out_specs=vm,
        scratch_shapes=[pltpu.VMEM((B, _SEL), jnp.float32),
                        pltpu.VMEM((B, state_dim), jnp.bfloat16)],
        compiler_params=pltpu.CompilerParams(
            dimension_semantics=("arbitrary",),
            vmem_limit_bytes=48 * 1024 * 1024),
    )(x0, cur_bt, w1x, w1c, w1t, b1, w2dt, b2dt)
